# baseline (speedup 1.0000x reference)
import functools

import jax
import jax.numpy as jnp
from jax import lax
from jax.experimental import pallas as pl
from jax.experimental.pallas import tpu as pltpu
from jax.experimental.pallas import tpu_sc as plsc

EMB = 64
HID = 64
V = 1000
N = 1024 * 200
NC = 2
NS = 16
NW = NC * NS
B_PER_W = N // NW
C = 40
STEPS = B_PER_W // C


def _table_body(emb_ref, wh_ref, bh_ref, wo_ref, bo_ref, out_ref):
    h = jnp.tanh(
        jnp.dot(emb_ref[...], wh_ref[...], preferred_element_type=jnp.float32)
        + bh_ref[...]
    )
    out_ref[...] = (
        jnp.dot(h, wo_ref[...], preferred_element_type=jnp.float32)
        + bo_ref[...]
    )


def _build_table(emb_table, W_h, b_h, W_o, b_o):
    return pl.pallas_call(
        _table_body,
        out_shape=jax.ShapeDtypeStruct((V, V), jnp.float32),
    )(emb_table, W_h, b_h.reshape(1, HID), W_o, b_o.reshape(1, V))


@functools.cache
def _make_gather_rows():
    mesh = plsc.VectorSubcoreMesh(core_axis_name="c", subcore_axis_name="s")

    @functools.partial(
        pl.kernel,
        mesh=mesh,
        compiler_params=pltpu.CompilerParams(use_tc_tiling_on_sc=False),
        out_type=jax.ShapeDtypeStruct((N, V), jnp.float32),
        scratch_types=[
            pltpu.VMEM((B_PER_W,), jnp.int32),
            pltpu.VMEM((C, V), jnp.float32),
            pltpu.SemaphoreType.DMA,
        ],
    )
    def _gather_rows(idx_hbm, table_hbm, out_hbm, idx_v, buf, sem):
        wid = lax.axis_index("s") * NC + lax.axis_index("c")
        base = wid * B_PER_W
        pltpu.sync_copy(idx_hbm.at[pl.ds(base, B_PER_W)], idx_v)

        def body(g, carry):
            start = g * C
            pltpu.async_copy(
                table_hbm.at[idx_v.at[pl.ds(start, C)]], buf, sem
            ).wait()
            pltpu.sync_copy(buf, out_hbm.at[pl.ds(base + start, C)])
            return carry

        lax.fori_loop(0, STEPS, body, 0)

    return _gather_rows


def kernel(x, emb_table, W_h, b_h, W_o, b_o):
    table = _build_table(emb_table, W_h, b_h, W_o, b_o)
    idx = x.reshape(-1).astype(jnp.int32)
    out = _make_gather_rows()(idx, table)
    return out.reshape(x.shape[0], x.shape[1], V)


# speedup vs baseline: 1.0391x; 1.0391x over previous
import functools

import jax
import jax.numpy as jnp
from jax import lax
from jax.experimental import pallas as pl
from jax.experimental.pallas import tpu as pltpu
from jax.experimental.pallas import tpu_sc as plsc

EMB = 64
HID = 64
V = 1000
N = 1024 * 200
NC = 2
NS = 16
NW = NC * NS
B_PER_W = N // NW
C = 40
STEPS = B_PER_W // C


def _table_body(emb_ref, wh_ref, bh_ref, wo_ref, bo_ref, out_ref):
    h = jnp.tanh(
        jnp.dot(emb_ref[...], wh_ref[...], preferred_element_type=jnp.float32)
        + bh_ref[...]
    )
    out_ref[...] = (
        jnp.dot(h, wo_ref[...], preferred_element_type=jnp.float32)
        + bo_ref[...]
    )


def _build_table(emb_table, W_h, b_h, W_o, b_o):
    return pl.pallas_call(
        _table_body,
        out_shape=jax.ShapeDtypeStruct((V, V), jnp.float32),
    )(emb_table, W_h, b_h.reshape(1, HID), W_o, b_o.reshape(1, V))


@functools.cache
def _make_gather_rows():
    mesh = plsc.VectorSubcoreMesh(core_axis_name="c", subcore_axis_name="s")

    @functools.partial(
        pl.kernel,
        mesh=mesh,
        compiler_params=pltpu.CompilerParams(use_tc_tiling_on_sc=False),
        out_type=jax.ShapeDtypeStruct((N, V), jnp.float32),
        scratch_types=[
            pltpu.VMEM((B_PER_W,), jnp.int32),
            pltpu.VMEM((C, V), jnp.float32),
            pltpu.VMEM((C, V), jnp.float32),
            pltpu.SemaphoreType.DMA,
            pltpu.SemaphoreType.DMA,
            pltpu.SemaphoreType.DMA,
            pltpu.SemaphoreType.DMA,
        ],
    )
    def _gather_rows(idx_hbm, table_hbm, out_hbm, idx_v, buf0, buf1,
                     sg0, sg1, sw0, sw1):
        wid = lax.axis_index("s") * NC + lax.axis_index("c")
        base = wid * B_PER_W
        pltpu.sync_copy(idx_hbm.at[pl.ds(base, B_PER_W)], idx_v)

        def gather(g, buf, sem):
            return pltpu.async_copy(
                table_hbm.at[idx_v.at[pl.ds(g * C, C)]], buf, sem
            )

        def write(g, buf, sem):
            return pltpu.async_copy(buf, out_hbm.at[pl.ds(base + g * C, C)], sem)

        half = STEPS // 2
        gather(0, buf0, sg0)

        def body(i, carry):
            g0 = i * 2

            @pl.when(i > 0)
            def _():
                pltpu.make_async_copy(buf1, out_hbm.at[pl.ds(base, C)], sw1).wait()

            gather(g0 + 1, buf1, sg1)
            pltpu.make_async_copy(
                table_hbm.at[idx_v.at[pl.ds(0, C)]], buf0, sg0
            ).wait()
            write(g0, buf0, sw0)
            pltpu.make_async_copy(
                table_hbm.at[idx_v.at[pl.ds(0, C)]], buf1, sg1
            ).wait()
            pltpu.make_async_copy(buf0, out_hbm.at[pl.ds(base, C)], sw0).wait()

            @pl.when(i < half - 1)
            def _():
                gather(g0 + 2, buf0, sg0)

            write(g0 + 1, buf1, sw1)
            return carry

        lax.fori_loop(0, half, body, 0)
        pltpu.make_async_copy(buf1, out_hbm.at[pl.ds(base, C)], sw1).wait()

    return _gather_rows


def kernel(x, emb_table, W_h, b_h, W_o, b_o):
    table = _build_table(emb_table, W_h, b_h, W_o, b_o)
    idx = x.reshape(-1).astype(jnp.int32)
    out = _make_gather_rows()(idx, table)
    return out.reshape(x.shape[0], x.shape[1], V)
